# baseline (speedup 1.0000x reference)
import functools

import jax
import jax.numpy as jnp
from jax import lax
from jax.experimental import pallas as pl
from jax.experimental.pallas import tpu as pltpu
from jax.experimental.pallas import tpu_sc as plsc

_REFILL_PROBABILITY = 0.5

_NC, _NS, _L = 2, 16, 16
_NW = _NC * _NS


def _take16(vec, idx):
    return vec.at[idx].get(mode="promise_in_bounds")


def _make_sc(B, A):
    PER_R = B // _NW
    CH_R = 8
    NCH = PER_R // CH_R
    AW = A // 4
    assert NCH >= 4 and NCH % 2 == 0 and PER_R % CH_R == 0 and A % 64 == 0

    mesh = plsc.VectorSubcoreMesh(core_axis_name="c", subcore_axis_name="s")

    def body(sup_h, cap_h, eq_h, wds_h, rand_h, tb_h, out_h,
             sup_v, cap_v, eq_v, wds_v, rand_v, out_v, tb_v,
             sin0, sin1, sot0, sot1):
        wid = lax.axis_index("c") * _NS + lax.axis_index("s")
        base = wid * PER_R
        pltpu.sync_copy(tb_h, tb_v)

        sins = (sin0, sin1)
        sots = (sot0, sot1)

        def in_copies(c, b):
            off = base + c * CH_R
            sem = sins[b]
            return (
                pltpu.make_async_copy(sup_h.at[pl.ds(off, CH_R)], sup_v.at[b], sem),
                pltpu.make_async_copy(cap_h.at[pl.ds(off, CH_R)], cap_v.at[b], sem),
                pltpu.make_async_copy(eq_h.at[pl.ds(off, CH_R)], eq_v.at[b], sem),
                pltpu.make_async_copy(rand_h.at[pl.ds(off, CH_R)], rand_v.at[b], sem),
                pltpu.make_async_copy(wds_h.at[pl.ds(off, CH_R)], wds_v.at[b], sem),
            )

        def out_copy(c, b):
            off = base + c * CH_R
            return pltpu.make_async_copy(out_v.at[b],
                                         out_h.at[pl.ds(off, CH_R)], sots[b])

        def issue_in(c, b):
            for cp in in_copies(c, b):
                cp.start()

        def wait_in(c, b):
            for cp in in_copies(c, b):
                cp.wait()

        def compute(b):
            tbv = tb_v[...]

            @plsc.parallel_loop(0, CH_R * (AW // 16), unroll=2)
            def inner(j):
                r = j >> 4
                g = j & 15
                w = wds_v[b, r, pl.ds(g * 16, 16)]
                for q in range(4):
                    col = q * AW + g * 16
                    mbit = w & (1 << (8 * q))
                    rnd = rand_v[b, r, pl.ds(col, 16)]
                    m = jnp.logical_and(mbit != 0,
                                        rnd < _REFILL_PROBABILITY)
                    bon = _take16(tbv, eq_v[b, r, pl.ds(col, 16)])
                    out_v[b, r, pl.ds(col, 16)] = jnp.where(
                        m, cap_v[b, r, pl.ds(col, 16)] + bon,
                        sup_v[b, r, pl.ds(col, 16)])

        issue_in(0, 0)
        issue_in(1, 1)
        for c in (0, 1):
            wait_in(c, c)
            compute(c)
            out_copy(c, c).start()
            issue_in(c + 2, c)

        def step(t, carry):
            j = 2 + 2 * t
            for b in range(2):
                c = j + b
                out_copy(c - 2, b).wait()
                wait_in(c, b)
                compute(b)
                out_copy(c, b).start()
                issue_in(c + 2, b)
            return carry

        if NCH > 4:
            lax.fori_loop(0, (NCH - 4) // 2, step, 0)

        for b in range(2):
            c = NCH - 2 + b
            out_copy(c - 2, b).wait()
            wait_in(c, b)
            compute(b)
            out_copy(c, b).start()
        for b in range(2):
            out_copy(NCH - 2 + b, b).wait()

    kern = functools.partial(
        pl.kernel,
        mesh=mesh,
        out_type=jax.ShapeDtypeStruct((B, A), jnp.float32),
        scratch_types=[
            pltpu.VMEM((2, CH_R, A), jnp.float32),
            pltpu.VMEM((2, CH_R, A), jnp.float32),
            pltpu.VMEM((2, CH_R, A), jnp.int32),
            pltpu.VMEM((2, CH_R, AW), jnp.int32),
            pltpu.VMEM((2, CH_R, A), jnp.float32),
            pltpu.VMEM((2, CH_R, A), jnp.float32),
            pltpu.VMEM((_L,), jnp.float32),
            pltpu.SemaphoreType.DMA,
            pltpu.SemaphoreType.DMA,
            pltpu.SemaphoreType.DMA,
            pltpu.SemaphoreType.DMA,
        ],
    )(body)
    return kern


def _pack_body(msk_ref, out_ref):
    AW = out_ref.shape[1]
    m = msk_ref[...]
    out_ref[...] = (m[:, :AW].astype(jnp.int32)
                    | (m[:, AW:2 * AW].astype(jnp.int32) << 8)
                    | (m[:, 2 * AW:3 * AW].astype(jnp.int32) << 16)
                    | (m[:, 3 * AW:].astype(jnp.int32) << 24))


def _pack_mask(refilled, block_b=1024):
    B, A = refilled.shape
    AW = A // 4
    return pl.pallas_call(
        _pack_body,
        grid=(B // block_b,),
        in_specs=[pl.BlockSpec((block_b, A), lambda i: (i, 0))],
        out_specs=pl.BlockSpec((block_b, AW), lambda i: (i, 0)),
        out_shape=jax.ShapeDtypeStruct((B, AW), jnp.int32),
        compiler_params=pltpu.CompilerParams(
            dimension_semantics=("arbitrary",),
        ),
    )(refilled)


def kernel(suppressants, capacity, equipment, refilled_suppressants,
           randomness_source, equipment_bonuses):
    B, A = suppressants.shape
    words = _pack_mask(refilled_suppressants)
    tb = jnp.concatenate(
        [equipment_bonuses.astype(jnp.float32),
         jnp.zeros((_L - equipment_bonuses.shape[0],), jnp.float32)])
    return _make_sc(B, A)(suppressants, capacity, equipment, words,
                          randomness_source, tb)


# speedup vs baseline: 1.2690x; 1.2690x over previous
import functools

import jax
import jax.numpy as jnp
from jax import lax
from jax.experimental import pallas as pl
from jax.experimental.pallas import tpu as pltpu
from jax.experimental.pallas import tpu_sc as plsc

_REFILL_PROBABILITY = 0.5

_NC, _NS, _L = 2, 16, 16
_NW = _NC * _NS


def _take16(vec, idx):
    return vec.at[idx].get(mode="promise_in_bounds")


def _make_sc(B, A):
    PER_R = B // _NW
    CH_R = 8
    NCH = PER_R // CH_R
    AW = A // 4
    assert NCH >= 4 and NCH % 2 == 0 and PER_R % CH_R == 0 and A % 64 == 0

    mesh = plsc.VectorSubcoreMesh(core_axis_name="c", subcore_axis_name="s")

    def body(sup_h, cap_h, eq_h, wds_h, rand_h, tb_h, out_h,
             sup_v, cap_v, eq_v, wds_v, rand_v, out_v, tb_v,
             sin0, sin1, sot0, sot1):
        wid = lax.axis_index("c") * _NS + lax.axis_index("s")
        base = wid * PER_R
        pltpu.sync_copy(tb_h, tb_v)

        sins = (sin0, sin1)
        sots = (sot0, sot1)

        def in_copies(c, b):
            off = base + c * CH_R
            sem = sins[b]
            return (
                pltpu.make_async_copy(sup_h.at[pl.ds(off, CH_R)], sup_v.at[b], sem),
                pltpu.make_async_copy(cap_h.at[pl.ds(off, CH_R)], cap_v.at[b], sem),
                pltpu.make_async_copy(eq_h.at[pl.ds(off, CH_R)], eq_v.at[b], sem),
                pltpu.make_async_copy(rand_h.at[pl.ds(off, CH_R)], rand_v.at[b], sem),
                pltpu.make_async_copy(wds_h.at[pl.ds(off, CH_R)], wds_v.at[b], sem),
            )

        def out_copy(c, b):
            off = base + c * CH_R
            return pltpu.make_async_copy(out_v.at[b],
                                         out_h.at[pl.ds(off, CH_R)], sots[b])

        def issue_in(c, b):
            for cp in in_copies(c, b):
                cp.start()

        def wait_in(c, b):
            for cp in in_copies(c, b):
                cp.wait()

        def compute(b):
            tbv = tb_v[...]

            @plsc.parallel_loop(0, CH_R * (AW // 16), unroll=2)
            def inner(j):
                r = j >> 4
                g = j & 15
                w = wds_v[b, r, pl.ds(g * 16, 16)]
                for q in range(4):
                    col = q * AW + g * 16
                    mbit = w & (1 << (8 * q))
                    rnd = rand_v[b, r, pl.ds(col, 16)]
                    m = jnp.logical_and(mbit != 0,
                                        rnd < _REFILL_PROBABILITY)
                    bon = _take16(tbv, eq_v[b, r, pl.ds(col, 16)])
                    out_v[b, r, pl.ds(col, 16)] = jnp.where(
                        m, cap_v[b, r, pl.ds(col, 16)] + bon,
                        sup_v[b, r, pl.ds(col, 16)])

        issue_in(0, 0)
        issue_in(1, 1)
        for c in (0, 1):
            wait_in(c, c)
            compute(c)
            out_copy(c, c).start()
            issue_in(c + 2, c)

        def step(t, carry):
            j = 2 + 2 * t
            for b in range(2):
                c = j + b
                out_copy(c - 2, b).wait()
                wait_in(c, b)
                compute(b)
                out_copy(c, b).start()
                issue_in(c + 2, b)
            return carry

        if NCH > 4:
            lax.fori_loop(0, (NCH - 4) // 2, step, 0)

        for b in range(2):
            c = NCH - 2 + b
            out_copy(c - 2, b).wait()
            wait_in(c, b)
            compute(b)
            out_copy(c, b).start()
        for b in range(2):
            out_copy(NCH - 2 + b, b).wait()

    kern = functools.partial(
        pl.kernel,
        mesh=mesh,
        out_type=jax.ShapeDtypeStruct((B, A), jnp.float32),
        scratch_types=[
            pltpu.VMEM((2, CH_R, A), jnp.float32),
            pltpu.VMEM((2, CH_R, A), jnp.float32),
            pltpu.VMEM((2, CH_R, A), jnp.int32),
            pltpu.VMEM((2, CH_R, AW), jnp.int32),
            pltpu.VMEM((2, CH_R, A), jnp.float32),
            pltpu.VMEM((2, CH_R, A), jnp.float32),
            pltpu.VMEM((_L,), jnp.float32),
            pltpu.SemaphoreType.DMA,
            pltpu.SemaphoreType.DMA,
            pltpu.SemaphoreType.DMA,
            pltpu.SemaphoreType.DMA,
        ],
    )(body)
    return kern


def kernel(suppressants, capacity, equipment, refilled_suppressants,
           randomness_source, equipment_bonuses):
    B, A = suppressants.shape
    AW = A // 4
    words = (refilled_suppressants[:, :AW].astype(jnp.int32)
             | (refilled_suppressants[:, AW:2 * AW].astype(jnp.int32) << 8)
             | (refilled_suppressants[:, 2 * AW:3 * AW].astype(jnp.int32) << 16)
             | (refilled_suppressants[:, 3 * AW:].astype(jnp.int32) << 24))
    tb = jnp.concatenate(
        [equipment_bonuses.astype(jnp.float32),
         jnp.zeros((_L - equipment_bonuses.shape[0],), jnp.float32)])
    return _make_sc(B, A)(suppressants, capacity, equipment, words,
                          randomness_source, tb)
